# baseline (speedup 1.0000x reference)
import jax
import jax.numpy as jnp
from jax.experimental import pallas as pl
from jax.experimental.pallas import tpu as pltpu

EPS_BN = 1e-5
EPS_NORM = 1e-12
N_CHUNKS = 2


def _norm_rows(v):
    ss = jnp.sum(v * v, axis=1, keepdims=True)
    return v * jax.lax.rsqrt(jnp.maximum(ss, EPS_NORM * EPS_NORM))


def _netvlad_kernel(length_ref, x_ref, halo_ref, taps_ref, shift1_ref,
                    w2_ref, bias2_ref, cent_ref, out_ref, vacc_ref,
                    aacc_ref):
    n, j = pl.program_id(0), pl.program_id(1)
    TB, C = x_ref.shape[2], x_ref.shape[3]
    K = cent_ref.shape[0]

    xn = _norm_rows(x_ref[0, 0])

    halo = _norm_rows(halo_ref[0, 0])
    prev = jnp.concatenate([halo[0:1, :], xn[:-1, :]], axis=0)
    nxt = jnp.concatenate([xn[1:, :], halo[1:2, :]], axis=0)
    h = (prev * taps_ref[0:1, :] + xn * taps_ref[1:2, :]
         + nxt * taps_ref[2:3, :] + shift1_ref[0:1, :])
    h = jnp.maximum(h, 0.0)

    s = jax.lax.dot_general(w2_ref[...], h, (((1,), (1,)), ((), ())),
                            preferred_element_type=jnp.float32)
    s = jnp.minimum(jnp.maximum(s + bias2_ref[...], 0.0), 80.0)

    t_idx = j * TB + jax.lax.broadcasted_iota(jnp.int32, (1, TB), 1)
    s = jnp.where(t_idx < length_ref[n], s, 0.0)
    e = jnp.exp(s)
    a = e * (1.0 / jnp.sum(e, axis=0, keepdims=True))

    vlad = jnp.dot(a, xn, preferred_element_type=jnp.float32)
    asum = jnp.sum(a, axis=1, keepdims=True)

    @pl.when(j == 0)
    def _():
        vacc_ref[...] = vlad
        aacc_ref[...] = asum

    @pl.when(j > 0)
    def _():
        vacc_ref[...] += vlad
        aacc_ref[...] += asum

    @pl.when(j == N_CHUNKS - 1)
    def _():
        v = vacc_ref[...] - aacc_ref[...] * cent_ref[...]
        n2 = jnp.sum(v * v, axis=1, keepdims=True)
        v = v * jax.lax.rsqrt(jnp.maximum(n2, EPS_NORM * EPS_NORM))
        g = jnp.sum(v * v)
        out_ref[0] = v * jax.lax.rsqrt(jnp.maximum(g, EPS_NORM * EPS_NORM))


def kernel(x_, conv1_w, bn1_gamma, bn1_beta, bn1_mean, bn1_var,
           conv2_w, conv2_b, bn2_gamma, bn2_beta, bn2_mean, bn2_var,
           centroids, length):
    N, T, C = x_.shape
    K = centroids.shape[0]
    J = N_CHUNKS
    TB = T // J

    scale1 = bn1_gamma * jax.lax.rsqrt(bn1_var + EPS_BN)
    shift1 = (bn1_beta - bn1_mean * scale1).reshape(1, C)
    taps = conv1_w[:, 0, :, 1].T * scale1[None, :]

    scale2 = bn2_gamma * jax.lax.rsqrt(bn2_var + EPS_BN)
    w2 = conv2_w[:, :, 0, 0] * scale2[:, None]
    bias2 = (conv2_b * scale2 + bn2_beta - bn2_mean * scale2).reshape(K, 1)

    zrow = jnp.zeros((N, 1, C), x_.dtype)
    hp = jnp.concatenate([zrow, x_[:, TB - 1::TB, :][:, :J - 1, :]], axis=1)
    hn = jnp.concatenate([x_[:, TB::TB, :][:, :J - 1, :], zrow], axis=1)
    halo = jnp.stack([hp, hn], axis=2)

    out = pl.pallas_call(
        _netvlad_kernel,
        grid=(N, J),
        in_specs=[
            pl.BlockSpec(memory_space=pltpu.SMEM),
            pl.BlockSpec((1, 1, TB, C), lambda n, j: (n, j, 0, 0)),
            pl.BlockSpec((1, 1, 2, C), lambda n, j: (n, j, 0, 0)),
            pl.BlockSpec((3, C), lambda n, j: (0, 0)),
            pl.BlockSpec((1, C), lambda n, j: (0, 0)),
            pl.BlockSpec((K, C), lambda n, j: (0, 0)),
            pl.BlockSpec((K, 1), lambda n, j: (0, 0)),
            pl.BlockSpec((K, C), lambda n, j: (0, 0)),
        ],
        out_specs=pl.BlockSpec((1, K, C), lambda n, j: (n, 0, 0)),
        out_shape=jax.ShapeDtypeStruct((N, K, C), jnp.float32),
        scratch_shapes=[
            pltpu.VMEM((K, C), jnp.float32),
            pltpu.VMEM((K, 1), jnp.float32),
        ],
        compiler_params=pltpu.CompilerParams(
            dimension_semantics=("arbitrary", "arbitrary"),
        ),
    )(length, x_.reshape(N, J, TB, C), halo, taps, shift1, w2, bias2,
      centroids)
    return out.reshape(N, K * C)


# speedup vs baseline: 1.2174x; 1.2174x over previous
import jax
import jax.numpy as jnp
from jax.experimental import pallas as pl
from jax.experimental.pallas import tpu as pltpu

EPS_BN = 1e-5
EPS_NORM = 1e-12


def _netvlad_kernel(length_ref, x_ref, taps_ref, shift1_ref, w2_ref,
                    bias2_ref, cent_ref, out_ref):
    n = pl.program_id(0)
    T, C = x_ref.shape[1], x_ref.shape[2]
    K = cent_ref.shape[0]

    x = x_ref[0]
    ss = jnp.sum(x * x, axis=1, keepdims=True)
    inv = jax.lax.rsqrt(jnp.maximum(ss, EPS_NORM * EPS_NORM))
    xn = x * inv

    zrow = jnp.zeros((1, C), jnp.float32)
    prev = jnp.concatenate([zrow, xn[:-1, :]], axis=0)
    nxt = jnp.concatenate([xn[1:, :], zrow], axis=0)
    h = (prev * taps_ref[0:1, :] + xn * taps_ref[1:2, :]
         + nxt * taps_ref[2:3, :] + shift1_ref[0:1, :])
    h = jnp.maximum(h, 0.0)

    s = jax.lax.dot_general(w2_ref[...], h, (((1,), (1,)), ((), ())),
                            preferred_element_type=jnp.float32)
    s = jnp.minimum(jnp.maximum(s + bias2_ref[...], 0.0), 80.0)

    t_idx = jax.lax.broadcasted_iota(jnp.int32, (1, T), 1)
    s = jnp.where(t_idx < length_ref[n], s, 0.0)
    e = jnp.exp(s)
    a = e * (1.0 / jnp.sum(e, axis=0, keepdims=True))

    vlad = jnp.dot(a, xn, preferred_element_type=jnp.float32)
    asum = jnp.sum(a, axis=1, keepdims=True)
    vlad = vlad - asum * cent_ref[...]

    n2 = jnp.sum(vlad * vlad, axis=1, keepdims=True)
    vlad = vlad * jax.lax.rsqrt(jnp.maximum(n2, EPS_NORM * EPS_NORM))
    g = jnp.sum(vlad * vlad)
    vlad = vlad * jax.lax.rsqrt(jnp.maximum(g, EPS_NORM * EPS_NORM))
    out_ref[0] = vlad


def kernel(x_, conv1_w, bn1_gamma, bn1_beta, bn1_mean, bn1_var,
           conv2_w, conv2_b, bn2_gamma, bn2_beta, bn2_mean, bn2_var,
           centroids, length):
    N, T, C = x_.shape
    K = centroids.shape[0]

    scale1 = bn1_gamma * jax.lax.rsqrt(bn1_var + EPS_BN)
    shift1 = (bn1_beta - bn1_mean * scale1).reshape(1, C)
    taps = conv1_w[:, 0, :, 1].T * scale1[None, :]

    scale2 = bn2_gamma * jax.lax.rsqrt(bn2_var + EPS_BN)
    w2 = conv2_w[:, :, 0, 0] * scale2[:, None]
    bias2 = (conv2_b * scale2 + bn2_beta - bn2_mean * scale2).reshape(K, 1)

    out = pl.pallas_call(
        _netvlad_kernel,
        grid=(N,),
        in_specs=[
            pl.BlockSpec(memory_space=pltpu.SMEM),
            pl.BlockSpec((1, T, C), lambda n: (n, 0, 0)),
            pl.BlockSpec((3, C), lambda n: (0, 0)),
            pl.BlockSpec((1, C), lambda n: (0, 0)),
            pl.BlockSpec((K, C), lambda n: (0, 0)),
            pl.BlockSpec((K, 1), lambda n: (0, 0)),
            pl.BlockSpec((K, C), lambda n: (0, 0)),
        ],
        out_specs=pl.BlockSpec((1, K, C), lambda n: (n, 0, 0)),
        out_shape=jax.ShapeDtypeStruct((N, K, C), jnp.float32),
        compiler_params=pltpu.CompilerParams(
            dimension_semantics=("arbitrary",),
        ),
    )(length, x_, taps, shift1, w2, bias2, centroids)
    return out.reshape(N, K * C)


# speedup vs baseline: 1.3942x; 1.1452x over previous
import jax
import jax.numpy as jnp
from jax.experimental import pallas as pl
from jax.experimental.pallas import tpu as pltpu

EPS_BN = 1e-5
EPS_NORM = 1e-12


def _netvlad_kernel(length_ref, x_ref, taps_ref, shift1_ref, w2_ref,
                    bias2_ref, cent_ref, out_ref):
    n = pl.program_id(0)
    T, C = x_ref.shape[1], x_ref.shape[2]
    K = cent_ref.shape[0]

    x = x_ref[0]
    ss = jnp.sum(x * x, axis=1, keepdims=True)
    inv = jax.lax.rsqrt(jnp.maximum(ss, EPS_NORM * EPS_NORM))
    xn = x * inv

    xb = xn.astype(jnp.bfloat16)
    tapsb = taps_ref[...].astype(jnp.bfloat16)
    sh1b = shift1_ref[...].astype(jnp.bfloat16)
    zrow = jnp.zeros((1, C), jnp.bfloat16)
    prev = jnp.concatenate([zrow, xb[:-1, :]], axis=0)
    nxt = jnp.concatenate([xb[1:, :], zrow], axis=0)
    h = (prev * tapsb[0:1, :] + xb * tapsb[1:2, :]
         + nxt * tapsb[2:3, :] + sh1b[0:1, :])
    h = jnp.maximum(h, 0.0)

    s = jax.lax.dot_general(w2_ref[...], h, (((1,), (1,)), ((), ())),
                            preferred_element_type=jnp.float32)
    s = jnp.minimum(jnp.maximum(s + bias2_ref[...], 0.0), 80.0)

    t_idx = jax.lax.broadcasted_iota(jnp.int32, (1, T), 1)
    s = jnp.where(t_idx < length_ref[n], s, 0.0)
    e = jnp.exp(s)
    a = e * (1.0 / jnp.sum(e, axis=0, keepdims=True))

    vlad = jnp.dot(a, xn, preferred_element_type=jnp.float32)
    asum = jnp.sum(a, axis=1, keepdims=True)
    vlad = vlad - asum * cent_ref[...]

    n2 = jnp.sum(vlad * vlad, axis=1, keepdims=True)
    vlad = vlad * jax.lax.rsqrt(jnp.maximum(n2, EPS_NORM * EPS_NORM))
    g = jnp.sum(vlad * vlad)
    vlad = vlad * jax.lax.rsqrt(jnp.maximum(g, EPS_NORM * EPS_NORM))
    out_ref[0] = vlad


def kernel(x_, conv1_w, bn1_gamma, bn1_beta, bn1_mean, bn1_var,
           conv2_w, conv2_b, bn2_gamma, bn2_beta, bn2_mean, bn2_var,
           centroids, length):
    N, T, C = x_.shape
    K = centroids.shape[0]

    scale1 = bn1_gamma * jax.lax.rsqrt(bn1_var + EPS_BN)
    shift1 = (bn1_beta - bn1_mean * scale1).reshape(1, C)
    taps = conv1_w[:, 0, :, 1].T * scale1[None, :]

    scale2 = bn2_gamma * jax.lax.rsqrt(bn2_var + EPS_BN)
    w2 = conv2_w[:, :, 0, 0] * scale2[:, None]
    bias2 = (conv2_b * scale2 + bn2_beta - bn2_mean * scale2).reshape(K, 1)

    out = pl.pallas_call(
        _netvlad_kernel,
        grid=(N,),
        in_specs=[
            pl.BlockSpec(memory_space=pltpu.SMEM),
            pl.BlockSpec((1, T, C), lambda n: (n, 0, 0)),
            pl.BlockSpec((3, C), lambda n: (0, 0)),
            pl.BlockSpec((1, C), lambda n: (0, 0)),
            pl.BlockSpec((K, C), lambda n: (0, 0)),
            pl.BlockSpec((K, 1), lambda n: (0, 0)),
            pl.BlockSpec((K, C), lambda n: (0, 0)),
        ],
        out_specs=pl.BlockSpec((1, K, C), lambda n: (n, 0, 0)),
        out_shape=jax.ShapeDtypeStruct((N, K, C), jnp.float32),
        compiler_params=pltpu.CompilerParams(
            dimension_semantics=("arbitrary",),
        ),
    )(length, x_, taps, shift1, w2, bias2, centroids)
    return out.reshape(N, K * C)
